# baseline (speedup 1.0000x reference)
import functools

import jax
import jax.numpy as jnp
from jax import lax
from jax.experimental import pallas as pl
from jax.experimental.pallas import tpu as pltpu
from jax.experimental.pallas import tpu_sc as plsc

NC = 2
NS = 16
NW = NC * NS
L = 16
HK = 8
NSLOT = 3


def _sc_body(per_w, x_hbm, ut_hbm, mt_hbm, out_hbm,
             xv, tiles_u, tiles_m, outv,
             sem_x, *sems):
    wid = lax.axis_index("s") * NC + lax.axis_index("c")
    base = wid * per_w
    n_rounds = per_w // HK

    pltpu.async_copy(x_hbm.at[pl.ds(2 * base, 2 * per_w)], xv, sem_x).wait()

    iota = lax.iota(jnp.int32, L)
    sem_u = sems[:NSLOT]
    sem_m = sems[NSLOT:]

    def fire(r, slot):
        rows2 = (iota + r * HK) * 2
        rows2 = jnp.minimum(rows2, 2 * per_w - 2)
        uvec = plsc.load_gather(xv, [rows2])
        mvec = plsc.load_gather(xv, [rows2 + 1])
        cu_all = lax.shift_right_logical(uvec, 7) * 128
        cm_all = lax.shift_right_logical(mvec, 7) * 128
        for jj in range(HK):
            cu = pl.multiple_of(cu_all[jj], 128)
            cm = pl.multiple_of(cm_all[jj], 128)
            du = tiles_u.at[slot * HK + jj]
            dm = tiles_m.at[slot * HK + jj]
            for h in range(2):
                hs = pl.ds(8 * h, 8)
                pltpu.async_copy(ut_hbm.at[hs, pl.ds(cu, 128)],
                                 du.at[hs], sem_u[slot])
                pltpu.async_copy(mt_hbm.at[hs, pl.ds(cm, 128)],
                                 dm.at[hs], sem_m[slot])

    def drain_compute(r, slot):
        for jj in range(HK):
            pltpu.make_async_copy(ut_hbm.at[:, pl.ds(0, 128)],
                                  tiles_u.at[slot * HK + jj],
                                  sem_u[slot]).wait()
            pltpu.make_async_copy(mt_hbm.at[:, pl.ds(0, 128)],
                                  tiles_m.at[slot * HK + jj],
                                  sem_m[slot]).wait()
        rows2 = (iota + r * HK) * 2
        rows2 = jnp.minimum(rows2, 2 * per_w - 2)
        uvec = plsc.load_gather(xv, [rows2])
        mvec = plsc.load_gather(xv, [rows2 + 1])
        ru_all = uvec & 127
        rm_all = mvec & 127
        out_slot = pl.ds(pl.multiple_of(r * HK, HK), L)
        acc = outv.at[out_slot][...]
        for jj in range(HK):
            jv = jnp.full((L,), slot * HK + jj, jnp.int32)
            ru = ru_all[jj] + jnp.zeros((L,), jnp.int32)
            rm = rm_all[jj] + jnp.zeros((L,), jnp.int32)
            u = plsc.load_gather(tiles_u, [jv, iota, ru])
            m = plsc.load_gather(tiles_m, [jv, iota, rm])
            acc = jnp.where(iota == jj, jnp.sum(u * m), acc)
        outv.at[out_slot][...] = acc

    fire(0, 0)
    fire(1, 1)

    @pl.loop(0, n_rounds - 1, step=NSLOT)
    def _(r):
        fire(r + 2, 2)
        drain_compute(r, 0)
        fire(r + 3, 0)
        drain_compute(r + 1, 1)

        @pl.when(r + 4 < n_rounds)
        def _():
            fire(r + 4, 1)

        drain_compute(r + 2, 2)

    drain_compute(n_rounds - 1, 0)

    pltpu.sync_copy(outv.at[pl.ds(0, per_w)], out_hbm.at[pl.ds(base, per_w)])


def kernel(x, U, M):
    batch = x.shape[0]
    per_w = batch // NW
    dim = U.shape[1]

    ut = U.T
    mt = M.T
    x_f = x.reshape(-1)

    mesh = plsc.VectorSubcoreMesh(core_axis_name="c", subcore_axis_name="s")
    cp = pltpu.CompilerParams(needs_layout_passes=False)
    k = pl.kernel(
        functools.partial(_sc_body, per_w),
        out_type=jax.ShapeDtypeStruct((batch,), jnp.float32),
        mesh=mesh,
        scratch_types=[
            pltpu.VMEM((2 * per_w,), jnp.int32),
            pltpu.VMEM((NSLOT * HK, dim, 128), jnp.float32),
            pltpu.VMEM((NSLOT * HK, dim, 128), jnp.float32),
            pltpu.VMEM((per_w + L,), jnp.float32),
            pltpu.SemaphoreType.DMA,
            pltpu.SemaphoreType.DMA,
            pltpu.SemaphoreType.DMA,
            pltpu.SemaphoreType.DMA,
            pltpu.SemaphoreType.DMA,
            pltpu.SemaphoreType.DMA,
            pltpu.SemaphoreType.DMA,
        ],
        compiler_params=cp,
    )
    out = k(x_f, ut, mt)
    return out.reshape(-1, 1)
